# baseline (speedup 1.0000x reference)
import functools

import jax
import jax.numpy as jnp
from jax import lax
from jax.experimental import pallas as pl
from jax.experimental.pallas import tpu as pltpu
from jax.experimental.pallas import tpu_sc as plsc

F32 = jnp.float32
LAT = 128


def _dot(a, b):
    return jnp.dot(a, b, preferred_element_type=F32)


def _r2(x):
    return x.reshape(1, -1)


def _mlp3_ln_body(x_ref, w1, b1, w2, b2, w3, b3, g, bln, o_ref):
    x = x_ref[...]
    h = jnp.maximum(_dot(x, w1[...]) + b1[...], 0.0)
    h = jnp.maximum(_dot(h, w2[...]) + b2[...], 0.0)
    h = _dot(h, w3[...]) + b3[...]
    m = jnp.mean(h, axis=-1, keepdims=True)
    v = jnp.mean((h - m) ** 2, axis=-1, keepdims=True)
    o_ref[...] = (h - m) / jnp.sqrt(v + 1e-5) * g[...] + bln[...]


def _mlp3_ln(x, ws, ln, br):
    (w1, b1), (w2, b2), (w3, b3) = ws
    g, b = ln
    mrows, fin = x.shape
    row = lambda i: (i, 0)
    zero = lambda i: (0, 0)
    return pl.pallas_call(
        _mlp3_ln_body,
        grid=(mrows // br,),
        in_specs=[
            pl.BlockSpec((br, fin), row),
            pl.BlockSpec((fin, LAT), zero),
            pl.BlockSpec((1, LAT), zero),
            pl.BlockSpec((LAT, LAT), zero),
            pl.BlockSpec((1, LAT), zero),
            pl.BlockSpec((LAT, LAT), zero),
            pl.BlockSpec((1, LAT), zero),
            pl.BlockSpec((1, LAT), zero),
            pl.BlockSpec((1, LAT), zero),
        ],
        out_specs=pl.BlockSpec((br, LAT), row),
        out_shape=jax.ShapeDtypeStruct((mrows, LAT), F32),
    )(x, w1, _r2(b1), w2, _r2(b2), w3, _r2(b3), _r2(g), _r2(b))


def _edge_body(e_ref, gs_ref, gd_ref, w1, b1, w2, b2, w3, b3, g, bln, o_ref):
    e = e_ref[...]
    h = jnp.maximum(_dot(e, w1[...]) + gs_ref[...] + gd_ref[...] + b1[...], 0.0)
    h = jnp.maximum(_dot(h, w2[...]) + b2[...], 0.0)
    h = _dot(h, w3[...]) + b3[...]
    m = jnp.mean(h, axis=-1, keepdims=True)
    v = jnp.mean((h - m) ** 2, axis=-1, keepdims=True)
    o_ref[...] = e + (h - m) / jnp.sqrt(v + 1e-5) * g[...] + bln[...]


def _edge_step(e, gs, gd, w1e, b1, w2, b2, w3, b3, ln, br):
    g, b = ln
    erows = e.shape[0]
    row = lambda i: (i, 0)
    zero = lambda i: (0, 0)
    return pl.pallas_call(
        _edge_body,
        grid=(erows // br,),
        in_specs=[
            pl.BlockSpec((br, LAT), row),
            pl.BlockSpec((br, LAT), row),
            pl.BlockSpec((br, LAT), row),
            pl.BlockSpec((LAT, LAT), zero),
            pl.BlockSpec((1, LAT), zero),
            pl.BlockSpec((LAT, LAT), zero),
            pl.BlockSpec((1, LAT), zero),
            pl.BlockSpec((LAT, LAT), zero),
            pl.BlockSpec((1, LAT), zero),
            pl.BlockSpec((1, LAT), zero),
            pl.BlockSpec((1, LAT), zero),
        ],
        out_specs=pl.BlockSpec((br, LAT), row),
        out_shape=jax.ShapeDtypeStruct((erows, LAT), F32),
    )(e, gs, gd, w1e, _r2(b1), w2, _r2(b2), w3, _r2(b3), _r2(g), _r2(b))


def _node_body(h_ref, a_ref, wh, wa, b1, w2, b2, w3, b3, g, bln, o_ref):
    hh = h_ref[...]
    x = jnp.maximum(_dot(hh, wh[...]) + _dot(a_ref[...], wa[...]) + b1[...], 0.0)
    x = jnp.maximum(_dot(x, w2[...]) + b2[...], 0.0)
    x = _dot(x, w3[...]) + b3[...]
    m = jnp.mean(x, axis=-1, keepdims=True)
    v = jnp.mean((x - m) ** 2, axis=-1, keepdims=True)
    o_ref[...] = hh + (x - m) / jnp.sqrt(v + 1e-5) * g[...] + bln[...]


def _node_body_proj(h_ref, a_ref, wh, wa, b1, w2, b2, w3, b3, g, bln, ws, wd,
                    o_ref, os_ref, od_ref):
    hh = h_ref[...]
    x = jnp.maximum(_dot(hh, wh[...]) + _dot(a_ref[...], wa[...]) + b1[...], 0.0)
    x = jnp.maximum(_dot(x, w2[...]) + b2[...], 0.0)
    x = _dot(x, w3[...]) + b3[...]
    m = jnp.mean(x, axis=-1, keepdims=True)
    v = jnp.mean((x - m) ** 2, axis=-1, keepdims=True)
    o = hh + (x - m) / jnp.sqrt(v + 1e-5) * g[...] + bln[...]
    o_ref[...] = o
    os_ref[...] = _dot(o, ws[...])
    od_ref[...] = _dot(o, wd[...])


def _node_step(h, agg, wh, wa, b1, w2, b2, w3, b3, ln, br, proj=None):
    g, b = ln
    nrows = h.shape[0]
    row = lambda i: (i, 0)
    zero = lambda i: (0, 0)
    specs = [
        pl.BlockSpec((br, LAT), row),
        pl.BlockSpec((br, LAT), row),
        pl.BlockSpec((LAT, LAT), zero),
        pl.BlockSpec((LAT, LAT), zero),
        pl.BlockSpec((1, LAT), zero),
        pl.BlockSpec((LAT, LAT), zero),
        pl.BlockSpec((1, LAT), zero),
        pl.BlockSpec((LAT, LAT), zero),
        pl.BlockSpec((1, LAT), zero),
        pl.BlockSpec((1, LAT), zero),
        pl.BlockSpec((1, LAT), zero),
    ]
    args = [h, agg, wh, wa, _r2(b1), w2, _r2(b2), w3, _r2(b3), _r2(g), _r2(b)]
    sds = jax.ShapeDtypeStruct((nrows, LAT), F32)
    if proj is None:
        return pl.pallas_call(
            _node_body,
            grid=(nrows // br,),
            in_specs=specs,
            out_specs=pl.BlockSpec((br, LAT), row),
            out_shape=sds,
        )(*args)
    ws, wd = proj
    return pl.pallas_call(
        _node_body_proj,
        grid=(nrows // br,),
        in_specs=specs + [pl.BlockSpec((LAT, LAT), zero),
                          pl.BlockSpec((LAT, LAT), zero)],
        out_specs=[pl.BlockSpec((br, LAT), row)] * 3,
        out_shape=[sds, sds, sds],
    )(*args, ws, wd)


def _mlp3_ln_proj_body(x_ref, w1, b1, w2, b2, w3, b3, g, bln, ws, wd,
                       o_ref, os_ref, od_ref):
    x = x_ref[...]
    h = jnp.maximum(_dot(x, w1[...]) + b1[...], 0.0)
    h = jnp.maximum(_dot(h, w2[...]) + b2[...], 0.0)
    h = _dot(h, w3[...]) + b3[...]
    m = jnp.mean(h, axis=-1, keepdims=True)
    v = jnp.mean((h - m) ** 2, axis=-1, keepdims=True)
    o = (h - m) / jnp.sqrt(v + 1e-5) * g[...] + bln[...]
    o_ref[...] = o
    os_ref[...] = _dot(o, ws[...])
    od_ref[...] = _dot(o, wd[...])


def _mlp3_ln_proj(x, ws_mlp, ln, proj, br):
    (w1, b1), (w2, b2), (w3, b3) = ws_mlp
    g, b = ln
    ws, wd = proj
    mrows, fin = x.shape
    row = lambda i: (i, 0)
    zero = lambda i: (0, 0)
    sds = jax.ShapeDtypeStruct((mrows, LAT), F32)
    return pl.pallas_call(
        _mlp3_ln_proj_body,
        grid=(mrows // br,),
        in_specs=[
            pl.BlockSpec((br, fin), row),
            pl.BlockSpec((fin, LAT), zero),
            pl.BlockSpec((1, LAT), zero),
            pl.BlockSpec((LAT, LAT), zero),
            pl.BlockSpec((1, LAT), zero),
            pl.BlockSpec((LAT, LAT), zero),
            pl.BlockSpec((1, LAT), zero),
            pl.BlockSpec((1, LAT), zero),
            pl.BlockSpec((1, LAT), zero),
            pl.BlockSpec((LAT, LAT), zero),
            pl.BlockSpec((LAT, LAT), zero),
        ],
        out_specs=[pl.BlockSpec((br, LAT), row)] * 3,
        out_shape=[sds, sds, sds],
    )(x, w1, _r2(b1), w2, _r2(b2), w3, _r2(b3), _r2(g), _r2(b), ws, wd)


def _head_body(do_ln, x_ref, w1, b1, w2, b2, w3, b3, g, bln, o_ref):
    x = x_ref[...]
    h = jnp.maximum(_dot(x, w1[...]) + b1[...], 0.0)
    h = jnp.maximum(_dot(h, w2[...]) + b2[...], 0.0)
    h = _dot(h, w3[...]) + b3[...]
    if do_ln:
        m = jnp.mean(h, axis=-1, keepdims=True)
        v = jnp.mean((h - m) ** 2, axis=-1, keepdims=True)
        h = (h - m) / jnp.sqrt(v + 1e-5) * g[...] + bln[...]
    o_ref[...] = h


def _head(x, ws, ln, do_ln, br):
    (w1, b1), (w2, b2), (w3, b3) = ws
    g, b = ln
    nrows = x.shape[0]
    fo = w3.shape[1]
    row = lambda i: (i, 0)
    zero = lambda i: (0, 0)
    return pl.pallas_call(
        functools.partial(_head_body, do_ln),
        grid=(nrows // br,),
        in_specs=[
            pl.BlockSpec((br, LAT), row),
            pl.BlockSpec((LAT, LAT), zero),
            pl.BlockSpec((1, LAT), zero),
            pl.BlockSpec((LAT, LAT), zero),
            pl.BlockSpec((1, LAT), zero),
            pl.BlockSpec((LAT, fo), zero),
            pl.BlockSpec((1, fo), zero),
            pl.BlockSpec((1, fo), zero),
            pl.BlockSpec((1, fo), zero),
        ],
        out_specs=pl.BlockSpec((br, fo), row),
        out_shape=jax.ShapeDtypeStruct((nrows, fo), F32),
    )(x, w1, _r2(b1), w2, _r2(b2), w3, _r2(b3), _r2(g), _r2(b))


def _topk3_matrix(d2, n_cols):
    it = lax.broadcasted_iota(jnp.int32, d2.shape, 1)
    acc = jnp.zeros(d2.shape, F32)
    wsum = jnp.zeros((d2.shape[0], 1), F32)
    for _ in range(3):
        m = jnp.min(d2, axis=1, keepdims=True)
        idx = jnp.min(jnp.where(d2 == m, it, n_cols), axis=1, keepdims=True)
        sel = it == idx
        w = 1.0 / jnp.maximum(m, 1e-16)
        acc = acc + jnp.where(sel, w, 0.0)
        wsum = wsum + w
        d2 = jnp.where(sel, 1e30, d2)
    return acc / wsum


def _geom_body(pm_ref, pp_ref, pmt_ref, ppt_ref, wd_ref, wu_ref):
    pm = pm_ref[...]
    pp = pp_ref[...]
    pmt = pmt_ref[...]
    ppt = ppt_ref[...]
    n_mesh = pm.shape[0]
    n_piv = pp.shape[0]
    dx = pp[:, 0:1] - pmt[0:1, :]
    dy = pp[:, 1:2] - pmt[1:2, :]
    wd_ref[...] = _topk3_matrix(dx * dx + dy * dy, n_mesh)
    ux = pm[:, 0:1] - ppt[0:1, :]
    uy = pm[:, 1:2] - ppt[1:2, :]
    wu_ref[...] = _topk3_matrix(ux * ux + uy * uy, n_piv)


def _geometry(pos_mesh, pos_piv):
    n_mesh = pos_mesh.shape[0]
    n_piv = pos_piv.shape[0]
    return pl.pallas_call(
        _geom_body,
        out_shape=[
            jax.ShapeDtypeStruct((n_piv, n_mesh), F32),
            jax.ShapeDtypeStruct((n_mesh, n_piv), F32),
        ],
    )(pos_mesh, pos_piv, pos_mesh.T, pos_piv.T)


def _mm_body(a_ref, b_ref, o_ref):
    o_ref[...] = _dot(a_ref[...], b_ref[...])


def _mm(a, b):
    return pl.pallas_call(
        _mm_body,
        out_shape=jax.ShapeDtypeStruct((a.shape[0], b.shape[1]), F32),
    )(a, b)


def _attn_body(piv_ref, pp_ref, wf, bf, wp, bp, win, binr, o_ref):
    x = piv_ref[0]
    n_piv = x.shape[0]
    emb = wf.shape[1]
    q = _dot(x, wf[...]) + bf[...] + _dot(pp_ref[...], wp[...]) + bp[...]
    qq = _dot(q, win[:, :emb]) + binr[:, :emb]
    kk = _dot(q, win[:, emb:2 * emb]) + binr[:, emb:2 * emb]
    n_heads = 4
    hd = emb // n_heads
    scale = 1.0 / (float(hd) ** 0.5)
    acc = jnp.zeros((n_piv, n_piv), F32)
    for hh in range(n_heads):
        qh = qq[:, hh * hd:(hh + 1) * hd]
        kh = kk[:, hh * hd:(hh + 1) * hd]
        s = lax.dot_general(qh, kh, (((1,), (1,)), ((), ())),
                            preferred_element_type=F32) * scale
        s = s - jnp.max(s, axis=-1, keepdims=True)
        es = jnp.exp(s)
        acc = acc + es / jnp.sum(es, axis=-1, keepdims=True)
    o_ref[0] = _dot(acc * (1.0 / n_heads), x)


def _attn(piv, pos_piv, wf, bf, wp, bp, win, binr):
    bsz, n_piv, fo = piv.shape
    emb = wf.shape[1]
    zero2 = lambda b: (0, 0)
    return pl.pallas_call(
        _attn_body,
        grid=(bsz,),
        in_specs=[
            pl.BlockSpec((1, n_piv, fo), lambda b: (b, 0, 0)),
            pl.BlockSpec((n_piv, 2), zero2),
            pl.BlockSpec((fo, emb), zero2),
            pl.BlockSpec((1, emb), zero2),
            pl.BlockSpec((2, emb), zero2),
            pl.BlockSpec((1, emb), zero2),
            pl.BlockSpec((emb, 3 * emb), zero2),
            pl.BlockSpec((1, 3 * emb), zero2),
        ],
        out_specs=pl.BlockSpec((1, n_piv, fo), lambda b: (b, 0, 0)),
        out_shape=jax.ShapeDtypeStruct((bsz, n_piv, fo), F32),
    )(piv, pos_piv, wf, _r2(bf), wp, _r2(bp), win, _r2(binr))


_C = 128
_SPG = 2
_G = _SPG * _C


def _sc_gather(hs, hd, src2d, dst2d):
    erows = src2d.shape[0] * _C
    nw = 32
    ept = erows // nw
    irows = ept // _C
    ngrp = ept // _G
    npair = ngrp // 2
    mesh = plsc.VectorSubcoreMesh(core_axis_name="c", subcore_axis_name="s")

    dt = hs.dtype

    @functools.partial(
        pl.kernel,
        mesh=mesh,
        out_type=[
            jax.ShapeDtypeStruct((erows, LAT), dt),
            jax.ShapeDtypeStruct((erows, LAT), dt),
        ],
        scratch_types=[
            pltpu.VMEM((irows, _C), jnp.int32),
            pltpu.VMEM((_G, LAT), dt),
            pltpu.VMEM((_G, LAT), dt),
            pltpu.VMEM((_G, LAT), dt),
            pltpu.SemaphoreType.DMA,
            pltpu.SemaphoreType.DMA,
            pltpu.SemaphoreType.DMA,
            pltpu.SemaphoreType.DMA,
            pltpu.SemaphoreType.DMA,
            pltpu.SemaphoreType.DMA,
        ],
    )
    def k(hs_h, hd_h, src_h, dst_h, gs_h, gd_h, idx_v, rows_0, rows_1, rows_2,
          semg_0, semg_1, semg_2, semw_0, semw_1, semw_2):
        c = lax.axis_index("c")
        s = lax.axis_index("s")
        wid = s * 2 + c
        base = wid * ept
        bufs = (rows_0, rows_1, rows_2)
        semg = (semg_0, semg_1, semg_2)
        semw = (semw_0, semw_1, semw_2)

        def one(table_h, ih, oh):
            pltpu.sync_copy(ih.at[pl.ds(wid * irows, irows)], idx_v)

            def fireg(g, b):
                for j in range(_SPG):
                    pltpu.async_copy(table_h.at[idx_v.at[_SPG * g + j]],
                                     bufs[b].at[pl.ds(j * _C, _C)], semg[b])

            def waitg(g, b):
                for j in range(_SPG):
                    pltpu.make_async_copy(
                        table_h.at[idx_v.at[_SPG * g + j]],
                        bufs[b].at[pl.ds(j * _C, _C)], semg[b]).wait()

            def firew(g, b):
                pltpu.async_copy(bufs[b], oh.at[pl.ds(base + g * _G, _G)],
                                 semw[b])

            def waitw(g, b):
                pltpu.make_async_copy(bufs[b],
                                      oh.at[pl.ds(base + g * _G, _G)],
                                      semw[b]).wait()

            fireg(0, 0)
            waitg(0, 0)
            firew(0, 0)
            fireg(1, 1)
            waitg(1, 1)
            firew(1, 1)
            fireg(2, 2)

            def body(i, _):
                g = 3 * i + 2
                for kk in range(3):
                    gg = g + kk
                    b = (2 + kk) % 3
                    waitg(gg, b)
                    firew(gg, b)
                    waitw(gg - 2, (b + 1) % 3)
                    fireg(gg + 1, (b + 1) % 3)
                return 0

            lax.fori_loop(0, (ngrp - 4) // 3, body, 0)
            b = (ngrp - 2) % 3
            waitg(ngrp - 2, b)
            firew(ngrp - 2, b)
            waitw(ngrp - 4, (b + 1) % 3)
            fireg(ngrp - 1, (b + 1) % 3)
            b = (ngrp - 1) % 3
            waitg(ngrp - 1, b)
            firew(ngrp - 1, b)
            for gg in (ngrp - 3, ngrp - 2, ngrp - 1):
                waitw(gg, gg % 3)

        one(hs_h, src_h, gs_h)
        one(hd_h, dst_h, gd_h)

    return k(hs, hd, src2d, dst2d)


def _sc_segsum(e_new, dst2d, zeros_blk, n_nodes):
    erows = e_new.shape[0]
    quarter = n_nodes // 4
    zrows = quarter // 16
    ept = erows // 16
    mesh = plsc.VectorSubcoreMesh(core_axis_name="c", subcore_axis_name="s")

    irows = ept // _C
    npair = (ept // _G) // 2

    @functools.partial(
        pl.kernel,
        mesh=mesh,
        out_type=jax.ShapeDtypeStruct((n_nodes, LAT), F32),
        scratch_types=[
            pltpu.VMEM((irows, _C), jnp.int32),
            pltpu.VMEM((irows, _C), jnp.int32),
            pltpu.VMEM((_G, LAT), F32),
            pltpu.VMEM((_G, LAT), F32),
            pltpu.VMEM_SHARED((quarter + 8, LAT), F32),
            pltpu.SemaphoreType.DMA,
            pltpu.SemaphoreType.DMA,
        ],
    )
    def k(e_h, dst_h, z_h, out_h, idx_v, adj_v, rows_a, rows_b, acc_sh,
          sem_a, sem_b):
        c = lax.axis_index("c")
        s = lax.axis_index("s")
        tbase = s * ept
        pltpu.sync_copy(dst_h.at[pl.ds(s * irows, irows)], idx_v)

        for p in range(2):
            lo = (c * 2 + p) * quarter
            pltpu.sync_copy(z_h.at[pl.ds(0, zrows)],
                            acc_sh.at[pl.ds(s * zrows, zrows)])

            @pl.when(s == 0)
            def _():
                pltpu.sync_copy(z_h.at[pl.ds(0, 8)],
                                acc_sh.at[pl.ds(quarter, 8)])

            def adj_body(r, _):
                for t in range(_C // 16):
                    v = idx_v[r, pl.ds(t * 16, 16)]
                    ok = (v >= lo) & (v < lo + quarter)
                    adj_v[r, pl.ds(t * 16, 16)] = jnp.where(ok, v - lo, quarter)
                return 0

            lax.fori_loop(0, irows, adj_body, 0)
            plsc.subcore_barrier()

            def fire(g, buf, sem):
                pltpu.async_copy(e_h.at[pl.ds(tbase + g * _G, _G)], buf, sem)

            def wait(g, buf, sem):
                pltpu.make_async_copy(e_h.at[pl.ds(tbase + g * _G, _G)],
                                      buf, sem).wait()

            def scat(g, buf):
                for j in range(_SPG):
                    pltpu.sync_copy(buf.at[pl.ds(j * _C, _C)],
                                    acc_sh.at[adj_v.at[_SPG * g + j]], add=True)

            fire(0, rows_a, sem_a)

            def body(i, _):
                ga = 2 * i
                fire(ga + 1, rows_b, sem_b)
                wait(ga, rows_a, sem_a)
                scat(ga, rows_a)
                fire(ga + 2, rows_a, sem_a)
                wait(ga + 1, rows_b, sem_b)
                scat(ga + 1, rows_b)
                return 0

            lax.fori_loop(0, npair - 1, body, 0)
            ga = 2 * (npair - 1)
            fire(ga + 1, rows_b, sem_b)
            wait(ga, rows_a, sem_a)
            scat(ga, rows_a)
            wait(ga + 1, rows_b, sem_b)
            scat(ga + 1, rows_b)

            plsc.subcore_barrier()
            pltpu.sync_copy(acc_sh.at[pl.ds(s * zrows, zrows)],
                            out_h.at[pl.ds(lo + s * zrows, zrows)])

    return k(e_new, dst2d, zeros_blk)


_BR_N = 1024
_BR_E = 1024


def kernel(node_attr, edge_index, edge_attr, position_mesh, position_pivotal,
           batch_size, params):
    src, dst = edge_index[0], edge_index[1]
    src2d = src.reshape(-1, _C)
    dst2d = dst.reshape(-1, _C)
    n_nodes = node_attr.shape[0]
    n_mesh = position_mesh.shape[0]
    n_piv = position_pivotal.shape[0]
    bsz = n_nodes // n_mesh
    zeros_blk = jnp.zeros((n_nodes // 32, LAT), F32)

    def run_mgn(p, x):
        def pw(st):
            w1 = st['edge_mlp'][0][0]
            return (w1[LAT:2 * LAT], w1[2 * LAT:3 * LAT])

        steps = p['steps']
        h, hs, hd = _mlp3_ln_proj(x, p['node_enc'], p['node_enc_ln'],
                                  pw(steps[0]), _BR_N)
        e = _mlp3_ln(edge_attr, p['edge_enc'], p['edge_enc_ln'], _BR_E)
        for i, st in enumerate(steps):
            (w1, b1), (w2, b2), (w3, b3) = st['edge_mlp']
            gs, gd = _sc_gather(hs, hd, src2d, dst2d)
            e = _edge_step(e, gs, gd, w1[:LAT], b1, w2, b2, w3, b3,
                           st['edge_ln'], _BR_E)
            agg = _sc_segsum(e, dst2d, zeros_blk, n_nodes)
            (wn1, bn1), (wn2, bn2), (wn3, bn3) = st['node_mlp']
            proj = pw(steps[i + 1]) if i + 1 < len(steps) else None
            out = _node_step(h, agg, wn1[:LAT], wn1[LAT:], bn1,
                             wn2, bn2, wn3, bn3, st['node_ln'], _BR_N,
                             proj=proj)
            if proj is None:
                h = out
            else:
                h, hs, hd = out
        return h

    h = run_mgn(params['enc'], node_attr)
    h3 = _head(h, params['enc']['node_dec'], params['pivotal_ln'], True, _BR_N)

    wdown, wup = _geometry(position_mesh, position_pivotal)
    hstk = h3.reshape(bsz, n_mesh, 3).transpose(1, 0, 2).reshape(n_mesh, 3 * bsz)
    piv_stk = _mm(wdown, hstk)
    piv = piv_stk.reshape(n_piv, bsz, 3).transpose(1, 0, 2)

    wf, bf = params['feat_proj']
    wp, bp = params['pos_proj']
    win, binr = params['mha_in']
    piv2 = _attn(piv, position_pivotal, wf, bf, wp, bp, win, binr)

    piv2_stk = piv2.transpose(1, 0, 2).reshape(n_piv, 3 * bsz)
    mesh_stk = _mm(wup, piv2_stk)
    h2 = mesh_stk.reshape(n_mesh, bsz, 3).transpose(1, 0, 2).reshape(n_nodes, 3)

    h4 = run_mgn(params['dec'], h2)
    return _head(h4, params['dec']['node_dec'], params['pivotal_ln'], False, _BR_N)


# speedup vs baseline: 1.0071x; 1.0071x over previous
import functools

import jax
import jax.numpy as jnp
from jax import lax
from jax.experimental import pallas as pl
from jax.experimental.pallas import tpu as pltpu
from jax.experimental.pallas import tpu_sc as plsc

F32 = jnp.float32
LAT = 128


def _dot(a, b):
    return jnp.dot(a, b, preferred_element_type=F32)


def _r2(x):
    return x.reshape(1, -1)


def _mlp3_ln_body(x_ref, w1, b1, w2, b2, w3, b3, g, bln, o_ref):
    x = x_ref[...]
    h = jnp.maximum(_dot(x, w1[...]) + b1[...], 0.0)
    h = jnp.maximum(_dot(h, w2[...]) + b2[...], 0.0)
    h = _dot(h, w3[...]) + b3[...]
    m = jnp.mean(h, axis=-1, keepdims=True)
    v = jnp.mean((h - m) ** 2, axis=-1, keepdims=True)
    o_ref[...] = (h - m) / jnp.sqrt(v + 1e-5) * g[...] + bln[...]


def _mlp3_ln(x, ws, ln, br):
    (w1, b1), (w2, b2), (w3, b3) = ws
    g, b = ln
    mrows, fin = x.shape
    row = lambda i: (i, 0)
    zero = lambda i: (0, 0)
    return pl.pallas_call(
        _mlp3_ln_body,
        grid=(mrows // br,),
        in_specs=[
            pl.BlockSpec((br, fin), row),
            pl.BlockSpec((fin, LAT), zero),
            pl.BlockSpec((1, LAT), zero),
            pl.BlockSpec((LAT, LAT), zero),
            pl.BlockSpec((1, LAT), zero),
            pl.BlockSpec((LAT, LAT), zero),
            pl.BlockSpec((1, LAT), zero),
            pl.BlockSpec((1, LAT), zero),
            pl.BlockSpec((1, LAT), zero),
        ],
        out_specs=pl.BlockSpec((br, LAT), row),
        out_shape=jax.ShapeDtypeStruct((mrows, LAT), F32),
    )(x, w1, _r2(b1), w2, _r2(b2), w3, _r2(b3), _r2(g), _r2(b))


def _edge_body(e_ref, gs_ref, gd_ref, w1, b1, w2, b2, w3, b3, g, bln, o_ref):
    e = e_ref[...]
    h = jnp.maximum(_dot(e, w1[...]) + gs_ref[...] + gd_ref[...] + b1[...], 0.0)
    h = jnp.maximum(_dot(h, w2[...]) + b2[...], 0.0)
    h = _dot(h, w3[...]) + b3[...]
    m = jnp.mean(h, axis=-1, keepdims=True)
    v = jnp.mean((h - m) ** 2, axis=-1, keepdims=True)
    o_ref[...] = e + (h - m) / jnp.sqrt(v + 1e-5) * g[...] + bln[...]


def _edge_step(e, gs, gd, w1e, b1, w2, b2, w3, b3, ln, br):
    g, b = ln
    erows = e.shape[0]
    row = lambda i: (i, 0)
    zero = lambda i: (0, 0)
    return pl.pallas_call(
        _edge_body,
        grid=(erows // br,),
        in_specs=[
            pl.BlockSpec((br, LAT), row),
            pl.BlockSpec((br, LAT), row),
            pl.BlockSpec((br, LAT), row),
            pl.BlockSpec((LAT, LAT), zero),
            pl.BlockSpec((1, LAT), zero),
            pl.BlockSpec((LAT, LAT), zero),
            pl.BlockSpec((1, LAT), zero),
            pl.BlockSpec((LAT, LAT), zero),
            pl.BlockSpec((1, LAT), zero),
            pl.BlockSpec((1, LAT), zero),
            pl.BlockSpec((1, LAT), zero),
        ],
        out_specs=pl.BlockSpec((br, LAT), row),
        out_shape=jax.ShapeDtypeStruct((erows, LAT), F32),
    )(e, gs, gd, w1e, _r2(b1), w2, _r2(b2), w3, _r2(b3), _r2(g), _r2(b))


def _node_body(h_ref, a_ref, wh, wa, b1, w2, b2, w3, b3, g, bln, o_ref):
    hh = h_ref[...]
    x = jnp.maximum(_dot(hh, wh[...]) + _dot(a_ref[...], wa[...]) + b1[...], 0.0)
    x = jnp.maximum(_dot(x, w2[...]) + b2[...], 0.0)
    x = _dot(x, w3[...]) + b3[...]
    m = jnp.mean(x, axis=-1, keepdims=True)
    v = jnp.mean((x - m) ** 2, axis=-1, keepdims=True)
    o_ref[...] = hh + (x - m) / jnp.sqrt(v + 1e-5) * g[...] + bln[...]


def _node_body_proj(h_ref, a_ref, wh, wa, b1, w2, b2, w3, b3, g, bln, ws, wd,
                    o_ref, os_ref, od_ref):
    hh = h_ref[...]
    x = jnp.maximum(_dot(hh, wh[...]) + _dot(a_ref[...], wa[...]) + b1[...], 0.0)
    x = jnp.maximum(_dot(x, w2[...]) + b2[...], 0.0)
    x = _dot(x, w3[...]) + b3[...]
    m = jnp.mean(x, axis=-1, keepdims=True)
    v = jnp.mean((x - m) ** 2, axis=-1, keepdims=True)
    o = hh + (x - m) / jnp.sqrt(v + 1e-5) * g[...] + bln[...]
    o_ref[...] = o
    os_ref[...] = _dot(o, ws[...])
    od_ref[...] = _dot(o, wd[...])


def _node_step(h, agg, wh, wa, b1, w2, b2, w3, b3, ln, br, proj=None):
    g, b = ln
    nrows = h.shape[0]
    row = lambda i: (i, 0)
    zero = lambda i: (0, 0)
    specs = [
        pl.BlockSpec((br, LAT), row),
        pl.BlockSpec((br, LAT), row),
        pl.BlockSpec((LAT, LAT), zero),
        pl.BlockSpec((LAT, LAT), zero),
        pl.BlockSpec((1, LAT), zero),
        pl.BlockSpec((LAT, LAT), zero),
        pl.BlockSpec((1, LAT), zero),
        pl.BlockSpec((LAT, LAT), zero),
        pl.BlockSpec((1, LAT), zero),
        pl.BlockSpec((1, LAT), zero),
        pl.BlockSpec((1, LAT), zero),
    ]
    args = [h, agg, wh, wa, _r2(b1), w2, _r2(b2), w3, _r2(b3), _r2(g), _r2(b)]
    sds = jax.ShapeDtypeStruct((nrows, LAT), F32)
    if proj is None:
        return pl.pallas_call(
            _node_body,
            grid=(nrows // br,),
            in_specs=specs,
            out_specs=pl.BlockSpec((br, LAT), row),
            out_shape=sds,
        )(*args)
    ws, wd = proj
    return pl.pallas_call(
        _node_body_proj,
        grid=(nrows // br,),
        in_specs=specs + [pl.BlockSpec((LAT, LAT), zero),
                          pl.BlockSpec((LAT, LAT), zero)],
        out_specs=[pl.BlockSpec((br, LAT), row)] * 3,
        out_shape=[sds, sds, sds],
    )(*args, ws, wd)


def _mlp3_ln_proj_body(x_ref, w1, b1, w2, b2, w3, b3, g, bln, ws, wd,
                       o_ref, os_ref, od_ref):
    x = x_ref[...]
    h = jnp.maximum(_dot(x, w1[...]) + b1[...], 0.0)
    h = jnp.maximum(_dot(h, w2[...]) + b2[...], 0.0)
    h = _dot(h, w3[...]) + b3[...]
    m = jnp.mean(h, axis=-1, keepdims=True)
    v = jnp.mean((h - m) ** 2, axis=-1, keepdims=True)
    o = (h - m) / jnp.sqrt(v + 1e-5) * g[...] + bln[...]
    o_ref[...] = o
    os_ref[...] = _dot(o, ws[...])
    od_ref[...] = _dot(o, wd[...])


def _mlp3_ln_proj(x, ws_mlp, ln, proj, br):
    (w1, b1), (w2, b2), (w3, b3) = ws_mlp
    g, b = ln
    ws, wd = proj
    mrows, fin = x.shape
    row = lambda i: (i, 0)
    zero = lambda i: (0, 0)
    sds = jax.ShapeDtypeStruct((mrows, LAT), F32)
    return pl.pallas_call(
        _mlp3_ln_proj_body,
        grid=(mrows // br,),
        in_specs=[
            pl.BlockSpec((br, fin), row),
            pl.BlockSpec((fin, LAT), zero),
            pl.BlockSpec((1, LAT), zero),
            pl.BlockSpec((LAT, LAT), zero),
            pl.BlockSpec((1, LAT), zero),
            pl.BlockSpec((LAT, LAT), zero),
            pl.BlockSpec((1, LAT), zero),
            pl.BlockSpec((1, LAT), zero),
            pl.BlockSpec((1, LAT), zero),
            pl.BlockSpec((LAT, LAT), zero),
            pl.BlockSpec((LAT, LAT), zero),
        ],
        out_specs=[pl.BlockSpec((br, LAT), row)] * 3,
        out_shape=[sds, sds, sds],
    )(x, w1, _r2(b1), w2, _r2(b2), w3, _r2(b3), _r2(g), _r2(b), ws, wd)


def _head_body(do_ln, x_ref, w1, b1, w2, b2, w3, b3, g, bln, o_ref):
    x = x_ref[...]
    h = jnp.maximum(_dot(x, w1[...]) + b1[...], 0.0)
    h = jnp.maximum(_dot(h, w2[...]) + b2[...], 0.0)
    h = _dot(h, w3[...]) + b3[...]
    if do_ln:
        m = jnp.mean(h, axis=-1, keepdims=True)
        v = jnp.mean((h - m) ** 2, axis=-1, keepdims=True)
        h = (h - m) / jnp.sqrt(v + 1e-5) * g[...] + bln[...]
    o_ref[...] = h


def _head(x, ws, ln, do_ln, br):
    (w1, b1), (w2, b2), (w3, b3) = ws
    g, b = ln
    nrows = x.shape[0]
    fo = w3.shape[1]
    row = lambda i: (i, 0)
    zero = lambda i: (0, 0)
    return pl.pallas_call(
        functools.partial(_head_body, do_ln),
        grid=(nrows // br,),
        in_specs=[
            pl.BlockSpec((br, LAT), row),
            pl.BlockSpec((LAT, LAT), zero),
            pl.BlockSpec((1, LAT), zero),
            pl.BlockSpec((LAT, LAT), zero),
            pl.BlockSpec((1, LAT), zero),
            pl.BlockSpec((LAT, fo), zero),
            pl.BlockSpec((1, fo), zero),
            pl.BlockSpec((1, fo), zero),
            pl.BlockSpec((1, fo), zero),
        ],
        out_specs=pl.BlockSpec((br, fo), row),
        out_shape=jax.ShapeDtypeStruct((nrows, fo), F32),
    )(x, w1, _r2(b1), w2, _r2(b2), w3, _r2(b3), _r2(g), _r2(b))


def _topk3_matrix(d2, n_cols):
    it = lax.broadcasted_iota(jnp.int32, d2.shape, 1)
    acc = jnp.zeros(d2.shape, F32)
    wsum = jnp.zeros((d2.shape[0], 1), F32)
    for _ in range(3):
        m = jnp.min(d2, axis=1, keepdims=True)
        idx = jnp.min(jnp.where(d2 == m, it, n_cols), axis=1, keepdims=True)
        sel = it == idx
        w = 1.0 / jnp.maximum(m, 1e-16)
        acc = acc + jnp.where(sel, w, 0.0)
        wsum = wsum + w
        d2 = jnp.where(sel, 1e30, d2)
    return acc / wsum


def _geom_body(pm_ref, pp_ref, pmt_ref, ppt_ref, wd_ref, wu_ref):
    pm = pm_ref[...]
    pp = pp_ref[...]
    pmt = pmt_ref[...]
    ppt = ppt_ref[...]
    n_mesh = pm.shape[0]
    n_piv = pp.shape[0]
    dx = pp[:, 0:1] - pmt[0:1, :]
    dy = pp[:, 1:2] - pmt[1:2, :]
    wd_ref[...] = _topk3_matrix(dx * dx + dy * dy, n_mesh)
    ux = pm[:, 0:1] - ppt[0:1, :]
    uy = pm[:, 1:2] - ppt[1:2, :]
    wu_ref[...] = _topk3_matrix(ux * ux + uy * uy, n_piv)


def _geometry(pos_mesh, pos_piv):
    n_mesh = pos_mesh.shape[0]
    n_piv = pos_piv.shape[0]
    return pl.pallas_call(
        _geom_body,
        out_shape=[
            jax.ShapeDtypeStruct((n_piv, n_mesh), F32),
            jax.ShapeDtypeStruct((n_mesh, n_piv), F32),
        ],
    )(pos_mesh, pos_piv, pos_mesh.T, pos_piv.T)


def _mm_body(a_ref, b_ref, o_ref):
    o_ref[...] = _dot(a_ref[...], b_ref[...])


def _mm(a, b):
    return pl.pallas_call(
        _mm_body,
        out_shape=jax.ShapeDtypeStruct((a.shape[0], b.shape[1]), F32),
    )(a, b)


def _attn_body(piv_ref, pp_ref, wf, bf, wp, bp, win, binr, o_ref):
    x = piv_ref[0]
    n_piv = x.shape[0]
    emb = wf.shape[1]
    q = _dot(x, wf[...]) + bf[...] + _dot(pp_ref[...], wp[...]) + bp[...]
    qq = _dot(q, win[:, :emb]) + binr[:, :emb]
    kk = _dot(q, win[:, emb:2 * emb]) + binr[:, emb:2 * emb]
    n_heads = 4
    hd = emb // n_heads
    scale = 1.0 / (float(hd) ** 0.5)
    acc = jnp.zeros((n_piv, n_piv), F32)
    for hh in range(n_heads):
        qh = qq[:, hh * hd:(hh + 1) * hd]
        kh = kk[:, hh * hd:(hh + 1) * hd]
        s = lax.dot_general(qh, kh, (((1,), (1,)), ((), ())),
                            preferred_element_type=F32) * scale
        s = s - jnp.max(s, axis=-1, keepdims=True)
        es = jnp.exp(s)
        acc = acc + es / jnp.sum(es, axis=-1, keepdims=True)
    o_ref[0] = _dot(acc * (1.0 / n_heads), x)


def _attn(piv, pos_piv, wf, bf, wp, bp, win, binr):
    bsz, n_piv, fo = piv.shape
    emb = wf.shape[1]
    zero2 = lambda b: (0, 0)
    return pl.pallas_call(
        _attn_body,
        grid=(bsz,),
        in_specs=[
            pl.BlockSpec((1, n_piv, fo), lambda b: (b, 0, 0)),
            pl.BlockSpec((n_piv, 2), zero2),
            pl.BlockSpec((fo, emb), zero2),
            pl.BlockSpec((1, emb), zero2),
            pl.BlockSpec((2, emb), zero2),
            pl.BlockSpec((1, emb), zero2),
            pl.BlockSpec((emb, 3 * emb), zero2),
            pl.BlockSpec((1, 3 * emb), zero2),
        ],
        out_specs=pl.BlockSpec((1, n_piv, fo), lambda b: (b, 0, 0)),
        out_shape=jax.ShapeDtypeStruct((bsz, n_piv, fo), F32),
    )(piv, pos_piv, wf, _r2(bf), wp, _r2(bp), win, _r2(binr))


_C = 128
_SPG = 2
_G = _SPG * _C


def _sc_gather(hs, hd, src2d, dst2d):
    erows = src2d.shape[0] * _C
    nw = 32
    ept = erows // nw
    irows = ept // _C
    ngrp = ept // _G
    npair = ngrp // 2
    mesh = plsc.VectorSubcoreMesh(core_axis_name="c", subcore_axis_name="s")

    dt = hs.dtype

    @functools.partial(
        pl.kernel,
        mesh=mesh,
        out_type=[
            jax.ShapeDtypeStruct((erows, LAT), dt),
            jax.ShapeDtypeStruct((erows, LAT), dt),
        ],
        scratch_types=[
            pltpu.VMEM((irows, _C), jnp.int32),
            pltpu.VMEM((_G, LAT), dt),
            pltpu.VMEM((_G, LAT), dt),
            pltpu.SemaphoreType.DMA,
            pltpu.SemaphoreType.DMA,
        ],
    )
    def k(hs_h, hd_h, src_h, dst_h, gs_h, gd_h, idx_v, rows_a, rows_b,
          sem_a, sem_b):
        c = lax.axis_index("c")
        s = lax.axis_index("s")
        wid = s * 2 + c
        base = wid * ept

        def one(table_h, ih, oh):
            pltpu.sync_copy(ih.at[pl.ds(wid * irows, irows)], idx_v)

            def fire(g, buf, sem):
                for j in range(_SPG):
                    pltpu.async_copy(table_h.at[idx_v.at[_SPG * g + j]],
                                     buf.at[pl.ds(j * _C, _C)], sem)

            def wait(g, buf, sem):
                for j in range(_SPG):
                    pltpu.make_async_copy(table_h.at[idx_v.at[_SPG * g + j]],
                                          buf.at[pl.ds(j * _C, _C)], sem).wait()

            fire(0, rows_a, sem_a)

            def body(i, _):
                ga = 2 * i
                fire(ga + 1, rows_b, sem_b)
                wait(ga, rows_a, sem_a)
                pltpu.sync_copy(rows_a, oh.at[pl.ds(base + ga * _G, _G)])
                fire(ga + 2, rows_a, sem_a)
                wait(ga + 1, rows_b, sem_b)
                pltpu.sync_copy(rows_b, oh.at[pl.ds(base + (ga + 1) * _G, _G)])
                return 0

            lax.fori_loop(0, npair - 1, body, 0)
            ga = 2 * (npair - 1)
            fire(ga + 1, rows_b, sem_b)
            wait(ga, rows_a, sem_a)
            pltpu.sync_copy(rows_a, oh.at[pl.ds(base + ga * _G, _G)])
            wait(ga + 1, rows_b, sem_b)
            pltpu.sync_copy(rows_b, oh.at[pl.ds(base + (ga + 1) * _G, _G)])

        one(hs_h, src_h, gs_h)
        one(hd_h, dst_h, gd_h)

    return k(hs, hd, src2d, dst2d)


def _sc_segsum(e_new, dst2d, zeros_blk, n_nodes):
    erows = e_new.shape[0]
    quarter = n_nodes // 4
    zrows = quarter // 16
    ept = erows // 16
    mesh = plsc.VectorSubcoreMesh(core_axis_name="c", subcore_axis_name="s")

    irows = ept // _C
    npair = (ept // _G) // 2

    @functools.partial(
        pl.kernel,
        mesh=mesh,
        out_type=jax.ShapeDtypeStruct((n_nodes, LAT), F32),
        scratch_types=[
            pltpu.VMEM((irows, _C), jnp.int32),
            pltpu.VMEM((irows, _C), jnp.int32),
            pltpu.VMEM((_G, LAT), F32),
            pltpu.VMEM((_G, LAT), F32),
            pltpu.VMEM_SHARED((quarter + 8, LAT), F32),
            pltpu.SemaphoreType.DMA,
            pltpu.SemaphoreType.DMA,
        ],
    )
    def k(e_h, dst_h, z_h, out_h, idx_v, adj_v, rows_a, rows_b, acc_sh,
          sem_a, sem_b):
        c = lax.axis_index("c")
        s = lax.axis_index("s")
        tbase = s * ept
        pltpu.sync_copy(dst_h.at[pl.ds(s * irows, irows)], idx_v)

        for p in range(2):
            lo = (c * 2 + p) * quarter
            pltpu.sync_copy(z_h.at[pl.ds(0, zrows)],
                            acc_sh.at[pl.ds(s * zrows, zrows)])

            @pl.when(s == 0)
            def _():
                pltpu.sync_copy(z_h.at[pl.ds(0, 8)],
                                acc_sh.at[pl.ds(quarter, 8)])

            def adj_body(r, _):
                for t in range(_C // 16):
                    v = idx_v[r, pl.ds(t * 16, 16)]
                    ok = (v >= lo) & (v < lo + quarter)
                    adj_v[r, pl.ds(t * 16, 16)] = jnp.where(ok, v - lo, quarter)
                return 0

            lax.fori_loop(0, irows, adj_body, 0)
            plsc.subcore_barrier()

            def fire(g, buf, sem):
                pltpu.async_copy(e_h.at[pl.ds(tbase + g * _G, _G)], buf, sem)

            def wait(g, buf, sem):
                pltpu.make_async_copy(e_h.at[pl.ds(tbase + g * _G, _G)],
                                      buf, sem).wait()

            def scat(g, buf):
                for j in range(_SPG):
                    pltpu.sync_copy(buf.at[pl.ds(j * _C, _C)],
                                    acc_sh.at[adj_v.at[_SPG * g + j]], add=True)

            fire(0, rows_a, sem_a)

            def body(i, _):
                ga = 2 * i
                fire(ga + 1, rows_b, sem_b)
                wait(ga, rows_a, sem_a)
                scat(ga, rows_a)
                fire(ga + 2, rows_a, sem_a)
                wait(ga + 1, rows_b, sem_b)
                scat(ga + 1, rows_b)
                return 0

            lax.fori_loop(0, npair - 1, body, 0)
            ga = 2 * (npair - 1)
            fire(ga + 1, rows_b, sem_b)
            wait(ga, rows_a, sem_a)
            scat(ga, rows_a)
            wait(ga + 1, rows_b, sem_b)
            scat(ga + 1, rows_b)

            plsc.subcore_barrier()
            pltpu.sync_copy(acc_sh.at[pl.ds(s * zrows, zrows)],
                            out_h.at[pl.ds(lo + s * zrows, zrows)])

    return k(e_new, dst2d, zeros_blk)


_BR_N = 1024
_BR_E = 1024


def kernel(node_attr, edge_index, edge_attr, position_mesh, position_pivotal,
           batch_size, params):
    src, dst = edge_index[0], edge_index[1]
    src2d = src.reshape(-1, _C)
    dst2d = dst.reshape(-1, _C)
    n_nodes = node_attr.shape[0]
    n_mesh = position_mesh.shape[0]
    n_piv = position_pivotal.shape[0]
    bsz = n_nodes // n_mesh
    zeros_blk = jnp.zeros((n_nodes // 32, LAT), F32)

    def run_mgn(p, x):
        def pw(st):
            w1 = st['edge_mlp'][0][0]
            return (w1[LAT:2 * LAT], w1[2 * LAT:3 * LAT])

        steps = p['steps']
        h, hs, hd = _mlp3_ln_proj(x, p['node_enc'], p['node_enc_ln'],
                                  pw(steps[0]), _BR_N)
        e = _mlp3_ln(edge_attr, p['edge_enc'], p['edge_enc_ln'], _BR_E)
        for i, st in enumerate(steps):
            (w1, b1), (w2, b2), (w3, b3) = st['edge_mlp']
            gs, gd = _sc_gather(hs, hd, src2d, dst2d)
            e = _edge_step(e, gs, gd, w1[:LAT], b1, w2, b2, w3, b3,
                           st['edge_ln'], _BR_E)
            agg = _sc_segsum(e, dst2d, zeros_blk, n_nodes)
            (wn1, bn1), (wn2, bn2), (wn3, bn3) = st['node_mlp']
            proj = pw(steps[i + 1]) if i + 1 < len(steps) else None
            out = _node_step(h, agg, wn1[:LAT], wn1[LAT:], bn1,
                             wn2, bn2, wn3, bn3, st['node_ln'], _BR_N,
                             proj=proj)
            if proj is None:
                h = out
            else:
                h, hs, hd = out
        return h

    h = run_mgn(params['enc'], node_attr)
    h3 = _head(h, params['enc']['node_dec'], params['pivotal_ln'], True, _BR_N)

    wdown, wup = _geometry(position_mesh, position_pivotal)
    hstk = h3.reshape(bsz, n_mesh, 3).transpose(1, 0, 2).reshape(n_mesh, 3 * bsz)
    piv_stk = _mm(wdown, hstk)
    piv = piv_stk.reshape(n_piv, bsz, 3).transpose(1, 0, 2)

    wf, bf = params['feat_proj']
    wp, bp = params['pos_proj']
    win, binr = params['mha_in']
    piv2 = _attn(piv, position_pivotal, wf, bf, wp, bp, win, binr)

    piv2_stk = piv2.transpose(1, 0, 2).reshape(n_piv, 3 * bsz)
    mesh_stk = _mm(wup, piv2_stk)
    h2 = mesh_stk.reshape(n_mesh, bsz, 3).transpose(1, 0, 2).reshape(n_nodes, 3)

    h4 = run_mgn(params['dec'], h2)
    return _head(h4, params['dec']['node_dec'], params['pivotal_ln'], False, _BR_N)


# speedup vs baseline: 1.1310x; 1.1230x over previous
import functools

import jax
import jax.numpy as jnp
from jax import lax
from jax.experimental import pallas as pl
from jax.experimental.pallas import tpu as pltpu
from jax.experimental.pallas import tpu_sc as plsc

F32 = jnp.float32
LAT = 128


def _dot(a, b):
    return jnp.dot(a, b, preferred_element_type=F32)


def _r2(x):
    return x.reshape(1, -1)


def _mlp3_ln_body(x_ref, w1, b1, w2, b2, w3, b3, g, bln, o_ref):
    x = x_ref[...]
    h = jnp.maximum(_dot(x, w1[...]) + b1[...], 0.0)
    h = jnp.maximum(_dot(h, w2[...]) + b2[...], 0.0)
    h = _dot(h, w3[...]) + b3[...]
    m = jnp.mean(h, axis=-1, keepdims=True)
    v = jnp.mean((h - m) ** 2, axis=-1, keepdims=True)
    o_ref[...] = (h - m) / jnp.sqrt(v + 1e-5) * g[...] + bln[...]


def _mlp3_ln(x, ws, ln, br):
    (w1, b1), (w2, b2), (w3, b3) = ws
    g, b = ln
    mrows, fin = x.shape
    row = lambda i: (i, 0)
    zero = lambda i: (0, 0)
    return pl.pallas_call(
        _mlp3_ln_body,
        grid=(mrows // br,),
        in_specs=[
            pl.BlockSpec((br, fin), row),
            pl.BlockSpec((fin, LAT), zero),
            pl.BlockSpec((1, LAT), zero),
            pl.BlockSpec((LAT, LAT), zero),
            pl.BlockSpec((1, LAT), zero),
            pl.BlockSpec((LAT, LAT), zero),
            pl.BlockSpec((1, LAT), zero),
            pl.BlockSpec((1, LAT), zero),
            pl.BlockSpec((1, LAT), zero),
        ],
        out_specs=pl.BlockSpec((br, LAT), row),
        out_shape=jax.ShapeDtypeStruct((mrows, LAT), F32),
    )(x, w1, _r2(b1), w2, _r2(b2), w3, _r2(b3), _r2(g), _r2(b))


def _edge_body(e_ref, gs_ref, gd_ref, w1, b1, w2, b2, w3, b3, g, bln, o_ref):
    e = e_ref[...]
    h = jnp.maximum(_dot(e, w1[...]) + gs_ref[...] + gd_ref[...] + b1[...], 0.0)
    h = jnp.maximum(_dot(h, w2[...]) + b2[...], 0.0)
    h = _dot(h, w3[...]) + b3[...]
    m = jnp.mean(h, axis=-1, keepdims=True)
    v = jnp.mean((h - m) ** 2, axis=-1, keepdims=True)
    o_ref[...] = e + (h - m) / jnp.sqrt(v + 1e-5) * g[...] + bln[...]


def _edge_step(e, gs, gd, w1e, b1, w2, b2, w3, b3, ln, br):
    g, b = ln
    erows = e.shape[0]
    row = lambda i: (i, 0)
    zero = lambda i: (0, 0)
    return pl.pallas_call(
        _edge_body,
        grid=(erows // br,),
        in_specs=[
            pl.BlockSpec((br, LAT), row),
            pl.BlockSpec((br, LAT), row),
            pl.BlockSpec((br, LAT), row),
            pl.BlockSpec((LAT, LAT), zero),
            pl.BlockSpec((1, LAT), zero),
            pl.BlockSpec((LAT, LAT), zero),
            pl.BlockSpec((1, LAT), zero),
            pl.BlockSpec((LAT, LAT), zero),
            pl.BlockSpec((1, LAT), zero),
            pl.BlockSpec((1, LAT), zero),
            pl.BlockSpec((1, LAT), zero),
        ],
        out_specs=pl.BlockSpec((br, LAT), row),
        out_shape=jax.ShapeDtypeStruct((erows, LAT), F32),
    )(e, gs, gd, w1e, _r2(b1), w2, _r2(b2), w3, _r2(b3), _r2(g), _r2(b))


def _node_body(h_ref, a_ref, wh, wa, b1, w2, b2, w3, b3, g, bln, o_ref):
    hh = h_ref[...]
    x = jnp.maximum(_dot(hh, wh[...]) + _dot(a_ref[...], wa[...]) + b1[...], 0.0)
    x = jnp.maximum(_dot(x, w2[...]) + b2[...], 0.0)
    x = _dot(x, w3[...]) + b3[...]
    m = jnp.mean(x, axis=-1, keepdims=True)
    v = jnp.mean((x - m) ** 2, axis=-1, keepdims=True)
    o_ref[...] = hh + (x - m) / jnp.sqrt(v + 1e-5) * g[...] + bln[...]


def _node_body_proj(h_ref, a_ref, wh, wa, b1, w2, b2, w3, b3, g, bln, ws, wd,
                    o_ref, os_ref, od_ref):
    hh = h_ref[...]
    x = jnp.maximum(_dot(hh, wh[...]) + _dot(a_ref[...], wa[...]) + b1[...], 0.0)
    x = jnp.maximum(_dot(x, w2[...]) + b2[...], 0.0)
    x = _dot(x, w3[...]) + b3[...]
    m = jnp.mean(x, axis=-1, keepdims=True)
    v = jnp.mean((x - m) ** 2, axis=-1, keepdims=True)
    o = hh + (x - m) / jnp.sqrt(v + 1e-5) * g[...] + bln[...]
    o_ref[...] = o
    os_ref[...] = _dot(o, ws[...])
    od_ref[...] = _dot(o, wd[...])


def _node_step(h, agg, wh, wa, b1, w2, b2, w3, b3, ln, br, proj=None):
    g, b = ln
    nrows = h.shape[0]
    row = lambda i: (i, 0)
    zero = lambda i: (0, 0)
    specs = [
        pl.BlockSpec((br, LAT), row),
        pl.BlockSpec((br, LAT), row),
        pl.BlockSpec((LAT, LAT), zero),
        pl.BlockSpec((LAT, LAT), zero),
        pl.BlockSpec((1, LAT), zero),
        pl.BlockSpec((LAT, LAT), zero),
        pl.BlockSpec((1, LAT), zero),
        pl.BlockSpec((LAT, LAT), zero),
        pl.BlockSpec((1, LAT), zero),
        pl.BlockSpec((1, LAT), zero),
        pl.BlockSpec((1, LAT), zero),
    ]
    args = [h, agg, wh, wa, _r2(b1), w2, _r2(b2), w3, _r2(b3), _r2(g), _r2(b)]
    sds = jax.ShapeDtypeStruct((nrows, LAT), F32)
    if proj is None:
        return pl.pallas_call(
            _node_body,
            grid=(nrows // br,),
            in_specs=specs,
            out_specs=pl.BlockSpec((br, LAT), row),
            out_shape=sds,
        )(*args)
    ws, wd = proj
    return pl.pallas_call(
        _node_body_proj,
        grid=(nrows // br,),
        in_specs=specs + [pl.BlockSpec((LAT, LAT), zero),
                          pl.BlockSpec((LAT, LAT), zero)],
        out_specs=[pl.BlockSpec((br, LAT), row)] * 3,
        out_shape=[sds, sds, sds],
    )(*args, ws, wd)


def _mlp3_ln_proj_body(x_ref, w1, b1, w2, b2, w3, b3, g, bln, ws, wd,
                       o_ref, os_ref, od_ref):
    x = x_ref[...]
    h = jnp.maximum(_dot(x, w1[...]) + b1[...], 0.0)
    h = jnp.maximum(_dot(h, w2[...]) + b2[...], 0.0)
    h = _dot(h, w3[...]) + b3[...]
    m = jnp.mean(h, axis=-1, keepdims=True)
    v = jnp.mean((h - m) ** 2, axis=-1, keepdims=True)
    o = (h - m) / jnp.sqrt(v + 1e-5) * g[...] + bln[...]
    o_ref[...] = o
    os_ref[...] = _dot(o, ws[...])
    od_ref[...] = _dot(o, wd[...])


def _mlp3_ln_proj(x, ws_mlp, ln, proj, br):
    (w1, b1), (w2, b2), (w3, b3) = ws_mlp
    g, b = ln
    ws, wd = proj
    mrows, fin = x.shape
    row = lambda i: (i, 0)
    zero = lambda i: (0, 0)
    sds = jax.ShapeDtypeStruct((mrows, LAT), F32)
    return pl.pallas_call(
        _mlp3_ln_proj_body,
        grid=(mrows // br,),
        in_specs=[
            pl.BlockSpec((br, fin), row),
            pl.BlockSpec((fin, LAT), zero),
            pl.BlockSpec((1, LAT), zero),
            pl.BlockSpec((LAT, LAT), zero),
            pl.BlockSpec((1, LAT), zero),
            pl.BlockSpec((LAT, LAT), zero),
            pl.BlockSpec((1, LAT), zero),
            pl.BlockSpec((1, LAT), zero),
            pl.BlockSpec((1, LAT), zero),
            pl.BlockSpec((LAT, LAT), zero),
            pl.BlockSpec((LAT, LAT), zero),
        ],
        out_specs=[pl.BlockSpec((br, LAT), row)] * 3,
        out_shape=[sds, sds, sds],
    )(x, w1, _r2(b1), w2, _r2(b2), w3, _r2(b3), _r2(g), _r2(b), ws, wd)


def _head_body(do_ln, x_ref, w1, b1, w2, b2, w3, b3, g, bln, o_ref):
    x = x_ref[...]
    h = jnp.maximum(_dot(x, w1[...]) + b1[...], 0.0)
    h = jnp.maximum(_dot(h, w2[...]) + b2[...], 0.0)
    h = _dot(h, w3[...]) + b3[...]
    if do_ln:
        m = jnp.mean(h, axis=-1, keepdims=True)
        v = jnp.mean((h - m) ** 2, axis=-1, keepdims=True)
        h = (h - m) / jnp.sqrt(v + 1e-5) * g[...] + bln[...]
    o_ref[...] = h


def _head(x, ws, ln, do_ln, br):
    (w1, b1), (w2, b2), (w3, b3) = ws
    g, b = ln
    nrows = x.shape[0]
    fo = w3.shape[1]
    row = lambda i: (i, 0)
    zero = lambda i: (0, 0)
    return pl.pallas_call(
        functools.partial(_head_body, do_ln),
        grid=(nrows // br,),
        in_specs=[
            pl.BlockSpec((br, LAT), row),
            pl.BlockSpec((LAT, LAT), zero),
            pl.BlockSpec((1, LAT), zero),
            pl.BlockSpec((LAT, LAT), zero),
            pl.BlockSpec((1, LAT), zero),
            pl.BlockSpec((LAT, fo), zero),
            pl.BlockSpec((1, fo), zero),
            pl.BlockSpec((1, fo), zero),
            pl.BlockSpec((1, fo), zero),
        ],
        out_specs=pl.BlockSpec((br, fo), row),
        out_shape=jax.ShapeDtypeStruct((nrows, fo), F32),
    )(x, w1, _r2(b1), w2, _r2(b2), w3, _r2(b3), _r2(g), _r2(b))


def _topk3_matrix(d2, n_cols):
    it = lax.broadcasted_iota(jnp.int32, d2.shape, 1)
    acc = jnp.zeros(d2.shape, F32)
    wsum = jnp.zeros((d2.shape[0], 1), F32)
    for _ in range(3):
        m = jnp.min(d2, axis=1, keepdims=True)
        idx = jnp.min(jnp.where(d2 == m, it, n_cols), axis=1, keepdims=True)
        sel = it == idx
        w = 1.0 / jnp.maximum(m, 1e-16)
        acc = acc + jnp.where(sel, w, 0.0)
        wsum = wsum + w
        d2 = jnp.where(sel, 1e30, d2)
    return acc / wsum


def _geom_body(pm_ref, pp_ref, pmt_ref, ppt_ref, wd_ref, wu_ref):
    pm = pm_ref[...]
    pp = pp_ref[...]
    pmt = pmt_ref[...]
    ppt = ppt_ref[...]
    n_mesh = pm.shape[0]
    n_piv = pp.shape[0]
    dx = pp[:, 0:1] - pmt[0:1, :]
    dy = pp[:, 1:2] - pmt[1:2, :]
    wd_ref[...] = _topk3_matrix(dx * dx + dy * dy, n_mesh)
    ux = pm[:, 0:1] - ppt[0:1, :]
    uy = pm[:, 1:2] - ppt[1:2, :]
    wu_ref[...] = _topk3_matrix(ux * ux + uy * uy, n_piv)


def _geometry(pos_mesh, pos_piv):
    n_mesh = pos_mesh.shape[0]
    n_piv = pos_piv.shape[0]
    return pl.pallas_call(
        _geom_body,
        out_shape=[
            jax.ShapeDtypeStruct((n_piv, n_mesh), F32),
            jax.ShapeDtypeStruct((n_mesh, n_piv), F32),
        ],
    )(pos_mesh, pos_piv, pos_mesh.T, pos_piv.T)


def _mm_body(a_ref, b_ref, o_ref):
    o_ref[...] = _dot(a_ref[...], b_ref[...])


def _mm(a, b):
    return pl.pallas_call(
        _mm_body,
        out_shape=jax.ShapeDtypeStruct((a.shape[0], b.shape[1]), F32),
    )(a, b)


def _attn_body(piv_ref, pp_ref, wf, bf, wp, bp, win, binr, o_ref):
    x = piv_ref[0]
    n_piv = x.shape[0]
    emb = wf.shape[1]
    q = _dot(x, wf[...]) + bf[...] + _dot(pp_ref[...], wp[...]) + bp[...]
    qq = _dot(q, win[:, :emb]) + binr[:, :emb]
    kk = _dot(q, win[:, emb:2 * emb]) + binr[:, emb:2 * emb]
    n_heads = 4
    hd = emb // n_heads
    scale = 1.0 / (float(hd) ** 0.5)
    acc = jnp.zeros((n_piv, n_piv), F32)
    for hh in range(n_heads):
        qh = qq[:, hh * hd:(hh + 1) * hd]
        kh = kk[:, hh * hd:(hh + 1) * hd]
        s = lax.dot_general(qh, kh, (((1,), (1,)), ((), ())),
                            preferred_element_type=F32) * scale
        s = s - jnp.max(s, axis=-1, keepdims=True)
        es = jnp.exp(s)
        acc = acc + es / jnp.sum(es, axis=-1, keepdims=True)
    o_ref[0] = _dot(acc * (1.0 / n_heads), x)


def _attn(piv, pos_piv, wf, bf, wp, bp, win, binr):
    bsz, n_piv, fo = piv.shape
    emb = wf.shape[1]
    zero2 = lambda b: (0, 0)
    return pl.pallas_call(
        _attn_body,
        grid=(bsz,),
        in_specs=[
            pl.BlockSpec((1, n_piv, fo), lambda b: (b, 0, 0)),
            pl.BlockSpec((n_piv, 2), zero2),
            pl.BlockSpec((fo, emb), zero2),
            pl.BlockSpec((1, emb), zero2),
            pl.BlockSpec((2, emb), zero2),
            pl.BlockSpec((1, emb), zero2),
            pl.BlockSpec((emb, 3 * emb), zero2),
            pl.BlockSpec((1, 3 * emb), zero2),
        ],
        out_specs=pl.BlockSpec((1, n_piv, fo), lambda b: (b, 0, 0)),
        out_shape=jax.ShapeDtypeStruct((bsz, n_piv, fo), F32),
    )(piv, pos_piv, wf, _r2(bf), wp, _r2(bp), win, _r2(binr))


_C = 128
_SPG = 2
_G = _SPG * _C


def _sc_gather(hs, hd, src2d, dst2d):
    erows = src2d.shape[0] * _C
    nw = 32
    ept = erows // nw
    irows = ept // _C
    ngrp = ept // _G
    npair = ngrp // 2
    mesh = plsc.VectorSubcoreMesh(core_axis_name="c", subcore_axis_name="s")

    dt = hs.dtype

    @functools.partial(
        pl.kernel,
        mesh=mesh,
        out_type=[
            jax.ShapeDtypeStruct((erows, LAT), dt),
            jax.ShapeDtypeStruct((erows, LAT), dt),
        ],
        scratch_types=[
            pltpu.VMEM((irows, _C), jnp.int32),
            pltpu.VMEM((_G, LAT), dt),
            pltpu.VMEM((_G, LAT), dt),
            pltpu.SemaphoreType.DMA,
            pltpu.SemaphoreType.DMA,
        ],
    )
    def k(hs_h, hd_h, src_h, dst_h, gs_h, gd_h, idx_v, rows_a, rows_b,
          sem_a, sem_b):
        c = lax.axis_index("c")
        s = lax.axis_index("s")
        wid = s * 2 + c
        base = wid * ept

        def one(table_h, ih, oh):
            pltpu.sync_copy(ih.at[pl.ds(wid * irows, irows)], idx_v)

            def fire(g, buf, sem):
                for j in range(_SPG):
                    pltpu.async_copy(table_h.at[idx_v.at[_SPG * g + j]],
                                     buf.at[pl.ds(j * _C, _C)], sem)

            def wait(g, buf, sem):
                for j in range(_SPG):
                    pltpu.make_async_copy(table_h.at[idx_v.at[_SPG * g + j]],
                                          buf.at[pl.ds(j * _C, _C)], sem).wait()

            fire(0, rows_a, sem_a)

            def body(i, _):
                ga = 2 * i
                fire(ga + 1, rows_b, sem_b)
                wait(ga, rows_a, sem_a)
                pltpu.sync_copy(rows_a, oh.at[pl.ds(base + ga * _G, _G)])
                fire(ga + 2, rows_a, sem_a)
                wait(ga + 1, rows_b, sem_b)
                pltpu.sync_copy(rows_b, oh.at[pl.ds(base + (ga + 1) * _G, _G)])
                return 0

            lax.fori_loop(0, npair - 1, body, 0)
            ga = 2 * (npair - 1)
            fire(ga + 1, rows_b, sem_b)
            wait(ga, rows_a, sem_a)
            pltpu.sync_copy(rows_a, oh.at[pl.ds(base + ga * _G, _G)])
            wait(ga + 1, rows_b, sem_b)
            pltpu.sync_copy(rows_b, oh.at[pl.ds(base + (ga + 1) * _G, _G)])

        one(hs_h, src_h, gs_h)
        one(hd_h, dst_h, gd_h)

    return k(hs, hd, src2d, dst2d)


def _sc_segsum(e_new, dst2d, zeros_blk, n_nodes):
    erows = e_new.shape[0]
    quarter = n_nodes // 4
    zrows = quarter // 16
    ept = erows // 16
    mesh = plsc.VectorSubcoreMesh(core_axis_name="c", subcore_axis_name="s")

    irows = ept // _C
    npair = (ept // _G) // 2

    @functools.partial(
        pl.kernel,
        mesh=mesh,
        out_type=jax.ShapeDtypeStruct((n_nodes, LAT), F32),
        scratch_types=[
            pltpu.VMEM((irows, _C), jnp.int32),
            pltpu.VMEM((irows, _C), jnp.int32),
            pltpu.VMEM((_G, LAT), F32),
            pltpu.VMEM((_G, LAT), F32),
            pltpu.VMEM_SHARED((quarter + 8, LAT), F32),
            pltpu.SemaphoreType.DMA,
            pltpu.SemaphoreType.DMA,
        ],
    )
    def k(e_h, dst_h, z_h, out_h, idx_v, adj_v, rows_a, rows_b, acc_sh,
          sem_a, sem_b):
        c = lax.axis_index("c")
        s = lax.axis_index("s")
        tbase = s * ept
        pltpu.sync_copy(dst_h.at[pl.ds(s * irows, irows)], idx_v)

        for p in range(2):
            lo = (c * 2 + p) * quarter
            pltpu.sync_copy(z_h.at[pl.ds(0, zrows)],
                            acc_sh.at[pl.ds(s * zrows, zrows)])

            @pl.when(s == 0)
            def _():
                pltpu.sync_copy(z_h.at[pl.ds(0, 8)],
                                acc_sh.at[pl.ds(quarter, 8)])

            def adj_body(r, _):
                for t in range(_C // 16):
                    v = idx_v[r, pl.ds(t * 16, 16)]
                    ok = (v >= lo) & (v < lo + quarter)
                    adj_v[r, pl.ds(t * 16, 16)] = jnp.where(ok, v - lo, quarter)
                return 0

            lax.fori_loop(0, irows, adj_body, 0)
            plsc.subcore_barrier()

            def fire(g, buf, sem):
                pltpu.async_copy(e_h.at[pl.ds(tbase + g * _G, _G)], buf, sem)

            def wait(g, buf, sem):
                pltpu.make_async_copy(e_h.at[pl.ds(tbase + g * _G, _G)],
                                      buf, sem).wait()

            def scat(g, buf):
                for j in range(_SPG):
                    pltpu.sync_copy(buf.at[pl.ds(j * _C, _C)],
                                    acc_sh.at[adj_v.at[_SPG * g + j]], add=True)

            fire(0, rows_a, sem_a)

            def body(i, _):
                ga = 2 * i
                fire(ga + 1, rows_b, sem_b)
                wait(ga, rows_a, sem_a)
                scat(ga, rows_a)
                fire(ga + 2, rows_a, sem_a)
                wait(ga + 1, rows_b, sem_b)
                scat(ga + 1, rows_b)
                return 0

            lax.fori_loop(0, npair - 1, body, 0)
            ga = 2 * (npair - 1)
            fire(ga + 1, rows_b, sem_b)
            wait(ga, rows_a, sem_a)
            scat(ga, rows_a)
            wait(ga + 1, rows_b, sem_b)
            scat(ga + 1, rows_b)

            plsc.subcore_barrier()
            pltpu.sync_copy(acc_sh.at[pl.ds(s * zrows, zrows)],
                            out_h.at[pl.ds(lo + s * zrows, zrows)])

    return k(e_new, dst2d, zeros_blk)


_BR_N = 2048
_BR_E = 2048


def kernel(node_attr, edge_index, edge_attr, position_mesh, position_pivotal,
           batch_size, params):
    src, dst = edge_index[0], edge_index[1]
    src2d = src.reshape(-1, _C)
    dst2d = dst.reshape(-1, _C)
    n_nodes = node_attr.shape[0]
    n_mesh = position_mesh.shape[0]
    n_piv = position_pivotal.shape[0]
    bsz = n_nodes // n_mesh
    zeros_blk = jnp.zeros((n_nodes // 32, LAT), F32)

    def run_mgn(p, x):
        def pw(st):
            w1 = st['edge_mlp'][0][0]
            return (w1[LAT:2 * LAT], w1[2 * LAT:3 * LAT])

        steps = p['steps']
        h, hs, hd = _mlp3_ln_proj(x, p['node_enc'], p['node_enc_ln'],
                                  pw(steps[0]), _BR_N)
        e = _mlp3_ln(edge_attr, p['edge_enc'], p['edge_enc_ln'], _BR_E)
        for i, st in enumerate(steps):
            (w1, b1), (w2, b2), (w3, b3) = st['edge_mlp']
            gs, gd = _sc_gather(hs, hd, src2d, dst2d)
            e = _edge_step(e, gs, gd, w1[:LAT], b1, w2, b2, w3, b3,
                           st['edge_ln'], _BR_E)
            agg = _sc_segsum(e, dst2d, zeros_blk, n_nodes)
            (wn1, bn1), (wn2, bn2), (wn3, bn3) = st['node_mlp']
            proj = pw(steps[i + 1]) if i + 1 < len(steps) else None
            out = _node_step(h, agg, wn1[:LAT], wn1[LAT:], bn1,
                             wn2, bn2, wn3, bn3, st['node_ln'], _BR_N,
                             proj=proj)
            if proj is None:
                h = out
            else:
                h, hs, hd = out
        return h

    h = run_mgn(params['enc'], node_attr)
    h3 = _head(h, params['enc']['node_dec'], params['pivotal_ln'], True, _BR_N)

    wdown, wup = _geometry(position_mesh, position_pivotal)
    hstk = h3.reshape(bsz, n_mesh, 3).transpose(1, 0, 2).reshape(n_mesh, 3 * bsz)
    piv_stk = _mm(wdown, hstk)
    piv = piv_stk.reshape(n_piv, bsz, 3).transpose(1, 0, 2)

    wf, bf = params['feat_proj']
    wp, bp = params['pos_proj']
    win, binr = params['mha_in']
    piv2 = _attn(piv, position_pivotal, wf, bf, wp, bp, win, binr)

    piv2_stk = piv2.transpose(1, 0, 2).reshape(n_piv, 3 * bsz)
    mesh_stk = _mm(wup, piv2_stk)
    h2 = mesh_stk.reshape(n_mesh, bsz, 3).transpose(1, 0, 2).reshape(n_nodes, 3)

    h4 = run_mgn(params['dec'], h2)
    return _head(h4, params['dec']['node_dec'], params['pivotal_ln'], False, _BR_N)


# speedup vs baseline: 1.1901x; 1.0523x over previous
import functools

import jax
import jax.numpy as jnp
from jax import lax
from jax.experimental import pallas as pl
from jax.experimental.pallas import tpu as pltpu
from jax.experimental.pallas import tpu_sc as plsc

F32 = jnp.float32
LAT = 128


def _dot(a, b):
    return jnp.dot(a, b, preferred_element_type=F32)


def _r2(x):
    return x.reshape(1, -1)


def _mlp3_ln_body(x_ref, w1, b1, w2, b2, w3, b3, g, bln, o_ref):
    x = x_ref[...]
    h = jnp.maximum(_dot(x, w1[...]) + b1[...], 0.0)
    h = jnp.maximum(_dot(h, w2[...]) + b2[...], 0.0)
    h = _dot(h, w3[...]) + b3[...]
    m = jnp.mean(h, axis=-1, keepdims=True)
    v = jnp.mean((h - m) ** 2, axis=-1, keepdims=True)
    o_ref[...] = (h - m) / jnp.sqrt(v + 1e-5) * g[...] + bln[...]


def _mlp3_ln(x, ws, ln, br):
    (w1, b1), (w2, b2), (w3, b3) = ws
    g, b = ln
    mrows, fin = x.shape
    row = lambda i: (i, 0)
    zero = lambda i: (0, 0)
    return pl.pallas_call(
        _mlp3_ln_body,
        grid=(mrows // br,),
        in_specs=[
            pl.BlockSpec((br, fin), row),
            pl.BlockSpec((fin, LAT), zero),
            pl.BlockSpec((1, LAT), zero),
            pl.BlockSpec((LAT, LAT), zero),
            pl.BlockSpec((1, LAT), zero),
            pl.BlockSpec((LAT, LAT), zero),
            pl.BlockSpec((1, LAT), zero),
            pl.BlockSpec((1, LAT), zero),
            pl.BlockSpec((1, LAT), zero),
        ],
        out_specs=pl.BlockSpec((br, LAT), row),
        out_shape=jax.ShapeDtypeStruct((mrows, LAT), F32),
    )(x, w1, _r2(b1), w2, _r2(b2), w3, _r2(b3), _r2(g), _r2(b))


def _edge_body(e_ref, gs_ref, gd_ref, w1, b1, w2, b2, w3, b3, g, bln, o_ref):
    e = e_ref[...]
    h = jnp.maximum(_dot(e, w1[...]) + gs_ref[...] + gd_ref[...] + b1[...], 0.0)
    h = jnp.maximum(_dot(h, w2[...]) + b2[...], 0.0)
    h = _dot(h, w3[...]) + b3[...]
    m = jnp.mean(h, axis=-1, keepdims=True)
    v = jnp.mean((h - m) ** 2, axis=-1, keepdims=True)
    o_ref[...] = e + (h - m) / jnp.sqrt(v + 1e-5) * g[...] + bln[...]


def _edge_step(e, gs, gd, w1e, b1, w2, b2, w3, b3, ln, br):
    g, b = ln
    erows = e.shape[0]
    row = lambda i: (i, 0)
    zero = lambda i: (0, 0)
    return pl.pallas_call(
        _edge_body,
        grid=(erows // br,),
        in_specs=[
            pl.BlockSpec((br, LAT), row),
            pl.BlockSpec((br, LAT), row),
            pl.BlockSpec((br, LAT), row),
            pl.BlockSpec((LAT, LAT), zero),
            pl.BlockSpec((1, LAT), zero),
            pl.BlockSpec((LAT, LAT), zero),
            pl.BlockSpec((1, LAT), zero),
            pl.BlockSpec((LAT, LAT), zero),
            pl.BlockSpec((1, LAT), zero),
            pl.BlockSpec((1, LAT), zero),
            pl.BlockSpec((1, LAT), zero),
        ],
        out_specs=pl.BlockSpec((br, LAT), row),
        out_shape=jax.ShapeDtypeStruct((erows, LAT), F32),
    )(e, gs, gd, w1e, _r2(b1), w2, _r2(b2), w3, _r2(b3), _r2(g), _r2(b))


def _node_body(h_ref, a_ref, wh, wa, b1, w2, b2, w3, b3, g, bln, o_ref):
    hh = h_ref[...]
    x = jnp.maximum(_dot(hh, wh[...]) + _dot(a_ref[...], wa[...]) + b1[...], 0.0)
    x = jnp.maximum(_dot(x, w2[...]) + b2[...], 0.0)
    x = _dot(x, w3[...]) + b3[...]
    m = jnp.mean(x, axis=-1, keepdims=True)
    v = jnp.mean((x - m) ** 2, axis=-1, keepdims=True)
    o_ref[...] = hh + (x - m) / jnp.sqrt(v + 1e-5) * g[...] + bln[...]


def _node_body_proj(h_ref, a_ref, wh, wa, b1, w2, b2, w3, b3, g, bln, ws, wd,
                    o_ref, os_ref, od_ref):
    hh = h_ref[...]
    x = jnp.maximum(_dot(hh, wh[...]) + _dot(a_ref[...], wa[...]) + b1[...], 0.0)
    x = jnp.maximum(_dot(x, w2[...]) + b2[...], 0.0)
    x = _dot(x, w3[...]) + b3[...]
    m = jnp.mean(x, axis=-1, keepdims=True)
    v = jnp.mean((x - m) ** 2, axis=-1, keepdims=True)
    o = hh + (x - m) / jnp.sqrt(v + 1e-5) * g[...] + bln[...]
    o_ref[...] = o
    os_ref[...] = _dot(o, ws[...])
    od_ref[...] = _dot(o, wd[...])


def _node_step(h, agg, wh, wa, b1, w2, b2, w3, b3, ln, br, proj=None):
    g, b = ln
    nrows = h.shape[0]
    row = lambda i: (i, 0)
    zero = lambda i: (0, 0)
    specs = [
        pl.BlockSpec((br, LAT), row),
        pl.BlockSpec((br, LAT), row),
        pl.BlockSpec((LAT, LAT), zero),
        pl.BlockSpec((LAT, LAT), zero),
        pl.BlockSpec((1, LAT), zero),
        pl.BlockSpec((LAT, LAT), zero),
        pl.BlockSpec((1, LAT), zero),
        pl.BlockSpec((LAT, LAT), zero),
        pl.BlockSpec((1, LAT), zero),
        pl.BlockSpec((1, LAT), zero),
        pl.BlockSpec((1, LAT), zero),
    ]
    args = [h, agg, wh, wa, _r2(b1), w2, _r2(b2), w3, _r2(b3), _r2(g), _r2(b)]
    sds = jax.ShapeDtypeStruct((nrows, LAT), F32)
    if proj is None:
        return pl.pallas_call(
            _node_body,
            grid=(nrows // br,),
            in_specs=specs,
            out_specs=pl.BlockSpec((br, LAT), row),
            out_shape=sds,
        )(*args)
    ws, wd = proj
    return pl.pallas_call(
        _node_body_proj,
        grid=(nrows // br,),
        in_specs=specs + [pl.BlockSpec((LAT, LAT), zero),
                          pl.BlockSpec((LAT, LAT), zero)],
        out_specs=[pl.BlockSpec((br, LAT), row)] * 3,
        out_shape=[sds, sds, sds],
    )(*args, ws, wd)


def _mlp3_ln_proj_body(x_ref, w1, b1, w2, b2, w3, b3, g, bln, ws, wd,
                       o_ref, os_ref, od_ref):
    x = x_ref[...]
    h = jnp.maximum(_dot(x, w1[...]) + b1[...], 0.0)
    h = jnp.maximum(_dot(h, w2[...]) + b2[...], 0.0)
    h = _dot(h, w3[...]) + b3[...]
    m = jnp.mean(h, axis=-1, keepdims=True)
    v = jnp.mean((h - m) ** 2, axis=-1, keepdims=True)
    o = (h - m) / jnp.sqrt(v + 1e-5) * g[...] + bln[...]
    o_ref[...] = o
    os_ref[...] = _dot(o, ws[...])
    od_ref[...] = _dot(o, wd[...])


def _mlp3_ln_proj(x, ws_mlp, ln, proj, br):
    (w1, b1), (w2, b2), (w3, b3) = ws_mlp
    g, b = ln
    ws, wd = proj
    mrows, fin = x.shape
    row = lambda i: (i, 0)
    zero = lambda i: (0, 0)
    sds = jax.ShapeDtypeStruct((mrows, LAT), F32)
    return pl.pallas_call(
        _mlp3_ln_proj_body,
        grid=(mrows // br,),
        in_specs=[
            pl.BlockSpec((br, fin), row),
            pl.BlockSpec((fin, LAT), zero),
            pl.BlockSpec((1, LAT), zero),
            pl.BlockSpec((LAT, LAT), zero),
            pl.BlockSpec((1, LAT), zero),
            pl.BlockSpec((LAT, LAT), zero),
            pl.BlockSpec((1, LAT), zero),
            pl.BlockSpec((1, LAT), zero),
            pl.BlockSpec((1, LAT), zero),
            pl.BlockSpec((LAT, LAT), zero),
            pl.BlockSpec((LAT, LAT), zero),
        ],
        out_specs=[pl.BlockSpec((br, LAT), row)] * 3,
        out_shape=[sds, sds, sds],
    )(x, w1, _r2(b1), w2, _r2(b2), w3, _r2(b3), _r2(g), _r2(b), ws, wd)


def _head_body(do_ln, x_ref, w1, b1, w2, b2, w3, b3, g, bln, o_ref):
    x = x_ref[...]
    h = jnp.maximum(_dot(x, w1[...]) + b1[...], 0.0)
    h = jnp.maximum(_dot(h, w2[...]) + b2[...], 0.0)
    h = _dot(h, w3[...]) + b3[...]
    if do_ln:
        m = jnp.mean(h, axis=-1, keepdims=True)
        v = jnp.mean((h - m) ** 2, axis=-1, keepdims=True)
        h = (h - m) / jnp.sqrt(v + 1e-5) * g[...] + bln[...]
    o_ref[...] = h


def _head(x, ws, ln, do_ln, br):
    (w1, b1), (w2, b2), (w3, b3) = ws
    g, b = ln
    nrows = x.shape[0]
    fo = w3.shape[1]
    row = lambda i: (i, 0)
    zero = lambda i: (0, 0)
    return pl.pallas_call(
        functools.partial(_head_body, do_ln),
        grid=(nrows // br,),
        in_specs=[
            pl.BlockSpec((br, LAT), row),
            pl.BlockSpec((LAT, LAT), zero),
            pl.BlockSpec((1, LAT), zero),
            pl.BlockSpec((LAT, LAT), zero),
            pl.BlockSpec((1, LAT), zero),
            pl.BlockSpec((LAT, fo), zero),
            pl.BlockSpec((1, fo), zero),
            pl.BlockSpec((1, fo), zero),
            pl.BlockSpec((1, fo), zero),
        ],
        out_specs=pl.BlockSpec((br, fo), row),
        out_shape=jax.ShapeDtypeStruct((nrows, fo), F32),
    )(x, w1, _r2(b1), w2, _r2(b2), w3, _r2(b3), _r2(g), _r2(b))


def _topk3_matrix(d2, n_cols):
    it = lax.broadcasted_iota(jnp.int32, d2.shape, 1)
    acc = jnp.zeros(d2.shape, F32)
    wsum = jnp.zeros((d2.shape[0], 1), F32)
    for _ in range(3):
        m = jnp.min(d2, axis=1, keepdims=True)
        idx = jnp.min(jnp.where(d2 == m, it, n_cols), axis=1, keepdims=True)
        sel = it == idx
        w = 1.0 / jnp.maximum(m, 1e-16)
        acc = acc + jnp.where(sel, w, 0.0)
        wsum = wsum + w
        d2 = jnp.where(sel, 1e30, d2)
    return acc / wsum


def _geom_body(pm_ref, pp_ref, pmt_ref, ppt_ref, wd_ref, wu_ref):
    pm = pm_ref[...]
    pp = pp_ref[...]
    pmt = pmt_ref[...]
    ppt = ppt_ref[...]
    n_mesh = pm.shape[0]
    n_piv = pp.shape[0]
    dx = pp[:, 0:1] - pmt[0:1, :]
    dy = pp[:, 1:2] - pmt[1:2, :]
    wd_ref[...] = _topk3_matrix(dx * dx + dy * dy, n_mesh)
    ux = pm[:, 0:1] - ppt[0:1, :]
    uy = pm[:, 1:2] - ppt[1:2, :]
    wu_ref[...] = _topk3_matrix(ux * ux + uy * uy, n_piv)


def _geometry(pos_mesh, pos_piv):
    n_mesh = pos_mesh.shape[0]
    n_piv = pos_piv.shape[0]
    return pl.pallas_call(
        _geom_body,
        out_shape=[
            jax.ShapeDtypeStruct((n_piv, n_mesh), F32),
            jax.ShapeDtypeStruct((n_mesh, n_piv), F32),
        ],
    )(pos_mesh, pos_piv, pos_mesh.T, pos_piv.T)


def _mm_body(a_ref, b_ref, o_ref):
    o_ref[...] = _dot(a_ref[...], b_ref[...])


def _mm(a, b):
    return pl.pallas_call(
        _mm_body,
        out_shape=jax.ShapeDtypeStruct((a.shape[0], b.shape[1]), F32),
    )(a, b)


def _attn_body(piv_ref, pp_ref, wf, bf, wp, bp, win, binr, o_ref):
    x = piv_ref[0]
    n_piv = x.shape[0]
    emb = wf.shape[1]
    q = _dot(x, wf[...]) + bf[...] + _dot(pp_ref[...], wp[...]) + bp[...]
    qq = _dot(q, win[:, :emb]) + binr[:, :emb]
    kk = _dot(q, win[:, emb:2 * emb]) + binr[:, emb:2 * emb]
    n_heads = 4
    hd = emb // n_heads
    scale = 1.0 / (float(hd) ** 0.5)
    acc = jnp.zeros((n_piv, n_piv), F32)
    for hh in range(n_heads):
        qh = qq[:, hh * hd:(hh + 1) * hd]
        kh = kk[:, hh * hd:(hh + 1) * hd]
        s = lax.dot_general(qh, kh, (((1,), (1,)), ((), ())),
                            preferred_element_type=F32) * scale
        s = s - jnp.max(s, axis=-1, keepdims=True)
        es = jnp.exp(s)
        acc = acc + es / jnp.sum(es, axis=-1, keepdims=True)
    o_ref[0] = _dot(acc * (1.0 / n_heads), x)


def _attn(piv, pos_piv, wf, bf, wp, bp, win, binr):
    bsz, n_piv, fo = piv.shape
    emb = wf.shape[1]
    zero2 = lambda b: (0, 0)
    return pl.pallas_call(
        _attn_body,
        grid=(bsz,),
        in_specs=[
            pl.BlockSpec((1, n_piv, fo), lambda b: (b, 0, 0)),
            pl.BlockSpec((n_piv, 2), zero2),
            pl.BlockSpec((fo, emb), zero2),
            pl.BlockSpec((1, emb), zero2),
            pl.BlockSpec((2, emb), zero2),
            pl.BlockSpec((1, emb), zero2),
            pl.BlockSpec((emb, 3 * emb), zero2),
            pl.BlockSpec((1, 3 * emb), zero2),
        ],
        out_specs=pl.BlockSpec((1, n_piv, fo), lambda b: (b, 0, 0)),
        out_shape=jax.ShapeDtypeStruct((bsz, n_piv, fo), F32),
    )(piv, pos_piv, wf, _r2(bf), wp, _r2(bp), win, _r2(binr))


_C = 128
_SPG = 2
_G = _SPG * _C


def _sc_gather(hs, hd, src2d, dst2d):
    erows = src2d.shape[0] * _C
    nw = 32
    ept = erows // nw
    irows = ept // _C
    ngrp = ept // _G
    npair = ngrp // 2
    mesh = plsc.VectorSubcoreMesh(core_axis_name="c", subcore_axis_name="s")

    dt = hs.dtype

    @functools.partial(
        pl.kernel,
        mesh=mesh,
        out_type=[
            jax.ShapeDtypeStruct((erows, LAT), dt),
            jax.ShapeDtypeStruct((erows, LAT), dt),
        ],
        scratch_types=[
            pltpu.VMEM((irows, _C), jnp.int32),
            pltpu.VMEM((_G, LAT), dt),
            pltpu.VMEM((_G, LAT), dt),
            pltpu.SemaphoreType.DMA,
            pltpu.SemaphoreType.DMA,
        ],
    )
    def k(hs_h, hd_h, src_h, dst_h, gs_h, gd_h, idx_v, rows_a, rows_b,
          sem_a, sem_b):
        c = lax.axis_index("c")
        s = lax.axis_index("s")
        wid = s * 2 + c
        base = wid * ept

        def one(table_h, ih, oh):
            pltpu.sync_copy(ih.at[pl.ds(wid * irows, irows)], idx_v)

            def fire(g, buf, sem):
                for j in range(_SPG):
                    pltpu.async_copy(table_h.at[idx_v.at[_SPG * g + j]],
                                     buf.at[pl.ds(j * _C, _C)], sem)

            def wait(g, buf, sem):
                for j in range(_SPG):
                    pltpu.make_async_copy(table_h.at[idx_v.at[_SPG * g + j]],
                                          buf.at[pl.ds(j * _C, _C)], sem).wait()

            fire(0, rows_a, sem_a)

            def body(i, _):
                ga = 2 * i
                fire(ga + 1, rows_b, sem_b)
                wait(ga, rows_a, sem_a)
                pltpu.sync_copy(rows_a, oh.at[pl.ds(base + ga * _G, _G)])
                fire(ga + 2, rows_a, sem_a)
                wait(ga + 1, rows_b, sem_b)
                pltpu.sync_copy(rows_b, oh.at[pl.ds(base + (ga + 1) * _G, _G)])
                return 0

            lax.fori_loop(0, npair - 1, body, 0)
            ga = 2 * (npair - 1)
            fire(ga + 1, rows_b, sem_b)
            wait(ga, rows_a, sem_a)
            pltpu.sync_copy(rows_a, oh.at[pl.ds(base + ga * _G, _G)])
            wait(ga + 1, rows_b, sem_b)
            pltpu.sync_copy(rows_b, oh.at[pl.ds(base + (ga + 1) * _G, _G)])

        one(hs_h, src_h, gs_h)
        one(hd_h, dst_h, gd_h)

    return k(hs, hd, src2d, dst2d)


def _sc_segsum(e_new, dst2d, zeros_blk, n_nodes):
    erows = e_new.shape[0]
    quarter = n_nodes // 4
    zrows = quarter // 16
    ept = erows // 16
    mesh = plsc.VectorSubcoreMesh(core_axis_name="c", subcore_axis_name="s")

    irows = ept // _C
    npair = (ept // _G) // 2

    @functools.partial(
        pl.kernel,
        mesh=mesh,
        out_type=jax.ShapeDtypeStruct((n_nodes, LAT), F32),
        scratch_types=[
            pltpu.VMEM((irows, _C), jnp.int32),
            pltpu.VMEM((irows, _C), jnp.int32),
            pltpu.VMEM((_G, LAT), F32),
            pltpu.VMEM((_G, LAT), F32),
            pltpu.VMEM_SHARED((quarter + 8, LAT), F32),
            pltpu.SemaphoreType.DMA,
            pltpu.SemaphoreType.DMA,
        ],
    )
    def k(e_h, dst_h, z_h, out_h, idx_v, adj_v, rows_a, rows_b, acc_sh,
          sem_a, sem_b):
        c = lax.axis_index("c")
        s = lax.axis_index("s")
        tbase = s * ept
        pltpu.sync_copy(dst_h.at[pl.ds(s * irows, irows)], idx_v)

        for p in range(2):
            lo = (c * 2 + p) * quarter
            pltpu.sync_copy(z_h.at[pl.ds(0, zrows)],
                            acc_sh.at[pl.ds(s * zrows, zrows)])

            @pl.when(s == 0)
            def _():
                pltpu.sync_copy(z_h.at[pl.ds(0, 8)],
                                acc_sh.at[pl.ds(quarter, 8)])

            def adj_body(r, _):
                for t in range(_C // 16):
                    v = idx_v[r, pl.ds(t * 16, 16)]
                    ok = (v >= lo) & (v < lo + quarter)
                    adj_v[r, pl.ds(t * 16, 16)] = jnp.where(ok, v - lo, quarter)
                return 0

            lax.fori_loop(0, irows, adj_body, 0)
            plsc.subcore_barrier()

            def fire(g, buf, sem):
                pltpu.async_copy(e_h.at[pl.ds(tbase + g * _G, _G)], buf, sem)

            def wait(g, buf, sem):
                pltpu.make_async_copy(e_h.at[pl.ds(tbase + g * _G, _G)],
                                      buf, sem).wait()

            def scat(g, buf):
                for j in range(_SPG):
                    pltpu.sync_copy(buf.at[pl.ds(j * _C, _C)],
                                    acc_sh.at[adj_v.at[_SPG * g + j]], add=True)

            fire(0, rows_a, sem_a)

            def body(i, _):
                ga = 2 * i
                fire(ga + 1, rows_b, sem_b)
                wait(ga, rows_a, sem_a)
                scat(ga, rows_a)
                fire(ga + 2, rows_a, sem_a)
                wait(ga + 1, rows_b, sem_b)
                scat(ga + 1, rows_b)
                return 0

            lax.fori_loop(0, npair - 1, body, 0)
            ga = 2 * (npair - 1)
            fire(ga + 1, rows_b, sem_b)
            wait(ga, rows_a, sem_a)
            scat(ga, rows_a)
            wait(ga + 1, rows_b, sem_b)
            scat(ga + 1, rows_b)

            plsc.subcore_barrier()
            pltpu.sync_copy(acc_sh.at[pl.ds(s * zrows, zrows)],
                            out_h.at[pl.ds(lo + s * zrows, zrows)])

    return k(e_new, dst2d, zeros_blk)


_BR_N = 4096
_BR_E = 4096


def kernel(node_attr, edge_index, edge_attr, position_mesh, position_pivotal,
           batch_size, params):
    src, dst = edge_index[0], edge_index[1]
    src2d = src.reshape(-1, _C)
    dst2d = dst.reshape(-1, _C)
    n_nodes = node_attr.shape[0]
    n_mesh = position_mesh.shape[0]
    n_piv = position_pivotal.shape[0]
    bsz = n_nodes // n_mesh
    zeros_blk = jnp.zeros((n_nodes // 32, LAT), F32)

    def run_mgn(p, x):
        def pw(st):
            w1 = st['edge_mlp'][0][0]
            return (w1[LAT:2 * LAT], w1[2 * LAT:3 * LAT])

        steps = p['steps']
        h, hs, hd = _mlp3_ln_proj(x, p['node_enc'], p['node_enc_ln'],
                                  pw(steps[0]), _BR_N)
        e = _mlp3_ln(edge_attr, p['edge_enc'], p['edge_enc_ln'], _BR_E)
        for i, st in enumerate(steps):
            (w1, b1), (w2, b2), (w3, b3) = st['edge_mlp']
            gs, gd = _sc_gather(hs, hd, src2d, dst2d)
            e = _edge_step(e, gs, gd, w1[:LAT], b1, w2, b2, w3, b3,
                           st['edge_ln'], _BR_E)
            agg = _sc_segsum(e, dst2d, zeros_blk, n_nodes)
            (wn1, bn1), (wn2, bn2), (wn3, bn3) = st['node_mlp']
            proj = pw(steps[i + 1]) if i + 1 < len(steps) else None
            out = _node_step(h, agg, wn1[:LAT], wn1[LAT:], bn1,
                             wn2, bn2, wn3, bn3, st['node_ln'], _BR_N,
                             proj=proj)
            if proj is None:
                h = out
            else:
                h, hs, hd = out
        return h

    h = run_mgn(params['enc'], node_attr)
    h3 = _head(h, params['enc']['node_dec'], params['pivotal_ln'], True, _BR_N)

    wdown, wup = _geometry(position_mesh, position_pivotal)
    hstk = h3.reshape(bsz, n_mesh, 3).transpose(1, 0, 2).reshape(n_mesh, 3 * bsz)
    piv_stk = _mm(wdown, hstk)
    piv = piv_stk.reshape(n_piv, bsz, 3).transpose(1, 0, 2)

    wf, bf = params['feat_proj']
    wp, bp = params['pos_proj']
    win, binr = params['mha_in']
    piv2 = _attn(piv, position_pivotal, wf, bf, wp, bp, win, binr)

    piv2_stk = piv2.transpose(1, 0, 2).reshape(n_piv, 3 * bsz)
    mesh_stk = _mm(wup, piv2_stk)
    h2 = mesh_stk.reshape(n_mesh, bsz, 3).transpose(1, 0, 2).reshape(n_nodes, 3)

    h4 = run_mgn(params['dec'], h2)
    return _head(h4, params['dec']['node_dec'], params['pivotal_ln'], False, _BR_N)
